# baseline (device time: 15662 ns/iter reference)
import jax
import jax.numpy as jnp
from jax import lax
from jax.experimental import pallas as pl
from jax.experimental.pallas import tpu as pltpu

N_DEV = 4
N_BLK = 2
_ORDER = (2, 1, 3)
_WAIT_ORDER = (1, 3, 2)


def kernel(partial, resid, gamma):
    _, m, n = partial.shape
    mq = m // N_DEV
    mb = mq // N_BLK
    gamma2 = gamma.reshape(1, n)

    def body(
        p_ref, r_ref, g_ref, o_ref,
        sendbuf, rs_comm, ag_send, ag_comm,
        rs_send_sems, rs_recv_sems, ag_send_sems, ag_recv_sems,
    ):
        me = lax.axis_index("i")

        barrier_sem = pltpu.get_barrier_semaphore()
        for o in range(1, N_DEV):
            pl.semaphore_signal(
                barrier_sem, inc=1,
                device_id=((me + o) % N_DEV,),
                device_id_type=pl.DeviceIdType.MESH,
            )
        sendbuf[...] = p_ref[0].astype(jnp.bfloat16)
        pl.semaphore_wait(barrier_sem, N_DEV - 1)

        rs_rdmas = {}
        for b in range(N_BLK):
            for o in _ORDER:
                dst = (me + o) % N_DEV
                rdma = pltpu.make_async_remote_copy(
                    src_ref=sendbuf.at[pl.ds(dst * mq + b * mb, mb), :],
                    dst_ref=rs_comm.at[b, o - 1],
                    send_sem=rs_send_sems.at[b, o - 1],
                    recv_sem=rs_recv_sems.at[b, o - 1],
                    device_id=(dst,),
                    device_id_type=pl.DeviceIdType.MESH,
                )
                rdma.start()
                rs_rdmas[b, o] = rdma

        ag_rdmas = {}
        for b in range(N_BLK):
            y = (
                p_ref[0, pl.ds(me * mq + b * mb, mb), :]
                + r_ref[pl.ds(me * mq + b * mb, mb), :]
            )
            for o in _WAIT_ORDER:
                rs_rdmas[b, o].wait_recv()
                y = y + rs_comm[b, o - 1].astype(jnp.float32)
            rms = jnp.sqrt(jnp.mean(y * y, axis=-1, keepdims=True) + 1e-6)
            mine = y / rms * g_ref[...]
            ag_send[pl.ds(b * mb, mb), :] = mine.astype(jnp.bfloat16)
            for o in _ORDER:
                dst = (me + o) % N_DEV
                rdma = pltpu.make_async_remote_copy(
                    src_ref=ag_send.at[pl.ds(b * mb, mb), :],
                    dst_ref=ag_comm.at[b, o - 1],
                    send_sem=ag_send_sems.at[b, o - 1],
                    recv_sem=ag_recv_sems.at[b, o - 1],
                    device_id=(dst,),
                    device_id_type=pl.DeviceIdType.MESH,
                )
                rdma.start()
                ag_rdmas[b, o] = rdma
            o_ref[pl.ds(me * mq + b * mb, mb), :] = mine

        for b in range(N_BLK):
            for o in _WAIT_ORDER:
                ag_rdmas[b, o].wait_recv()
                src_pos = (me - o) % N_DEV
                o_ref[pl.ds(src_pos * mq + b * mb, mb), :] = (
                    ag_comm[b, o - 1].astype(jnp.float32)
                )

        for b in range(N_BLK):
            for o in _ORDER:
                rs_rdmas[b, o].wait_send()
                ag_rdmas[b, o].wait_send()

    return pl.pallas_call(
        body,
        out_shape=jax.ShapeDtypeStruct((m, n), jnp.float32),
        in_specs=[pl.BlockSpec(memory_space=pltpu.VMEM)] * 3,
        out_specs=pl.BlockSpec(memory_space=pltpu.VMEM),
        scratch_shapes=[
            pltpu.VMEM((m, n), jnp.bfloat16),
            pltpu.VMEM((N_BLK, N_DEV - 1, mb, n), jnp.bfloat16),
            pltpu.VMEM((mq, n), jnp.bfloat16),
            pltpu.VMEM((N_BLK, N_DEV - 1, mb, n), jnp.bfloat16),
            pltpu.SemaphoreType.DMA((N_BLK, N_DEV - 1)),
            pltpu.SemaphoreType.DMA((N_BLK, N_DEV - 1)),
            pltpu.SemaphoreType.DMA((N_BLK, N_DEV - 1)),
            pltpu.SemaphoreType.DMA((N_BLK, N_DEV - 1)),
        ],
        compiler_params=pltpu.CompilerParams(collective_id=0),
    )(partial, resid, gamma2)


# device time: 15455 ns/iter; 1.0134x vs baseline; 1.0134x over previous
import jax
import jax.numpy as jnp
from jax import lax
from jax.experimental import pallas as pl
from jax.experimental.pallas import tpu as pltpu

N_DEV = 4
N_BLK = 4
_ORDER = (2, 1, 3)
_WAIT_ORDER = (1, 3, 2)


def kernel(partial, resid, gamma):
    _, m, n = partial.shape
    mq = m // N_DEV
    mb = mq // N_BLK
    gamma2 = gamma.reshape(1, n)

    def body(
        p_ref, r_ref, g_ref, o_ref,
        sendbuf, rs_comm, ag_send, ag_comm,
        rs_send_sems, rs_recv_sems, ag_send_sems, ag_recv_sems,
    ):
        me = lax.axis_index("i")

        barrier_sem = pltpu.get_barrier_semaphore()
        for o in range(1, N_DEV):
            pl.semaphore_signal(
                barrier_sem, inc=1,
                device_id=((me + o) % N_DEV,),
                device_id_type=pl.DeviceIdType.MESH,
            )
        sendbuf[...] = p_ref[0].astype(jnp.bfloat16)
        pl.semaphore_wait(barrier_sem, N_DEV - 1)

        rs_rdmas = {}
        for b in range(N_BLK):
            for o in _ORDER:
                dst = (me + o) % N_DEV
                rdma = pltpu.make_async_remote_copy(
                    src_ref=sendbuf.at[pl.ds(dst * mq + b * mb, mb), :],
                    dst_ref=rs_comm.at[b, o - 1],
                    send_sem=rs_send_sems.at[b, o - 1],
                    recv_sem=rs_recv_sems.at[b, o - 1],
                    device_id=(dst,),
                    device_id_type=pl.DeviceIdType.MESH,
                )
                rdma.start()
                rs_rdmas[b, o] = rdma

        ag_rdmas = {}
        for b in range(N_BLK):
            y = (
                p_ref[0, pl.ds(me * mq + b * mb, mb), :]
                + r_ref[pl.ds(me * mq + b * mb, mb), :]
            )
            for o in _WAIT_ORDER:
                rs_rdmas[b, o].wait_recv()
                y = y + rs_comm[b, o - 1].astype(jnp.float32)
            rms = jnp.sqrt(jnp.mean(y * y, axis=-1, keepdims=True) + 1e-6)
            mine = y / rms * g_ref[...]
            ag_send[pl.ds(b * mb, mb), :] = mine.astype(jnp.bfloat16)
            for o in _ORDER:
                dst = (me + o) % N_DEV
                rdma = pltpu.make_async_remote_copy(
                    src_ref=ag_send.at[pl.ds(b * mb, mb), :],
                    dst_ref=ag_comm.at[b, o - 1],
                    send_sem=ag_send_sems.at[b, o - 1],
                    recv_sem=ag_recv_sems.at[b, o - 1],
                    device_id=(dst,),
                    device_id_type=pl.DeviceIdType.MESH,
                )
                rdma.start()
                ag_rdmas[b, o] = rdma
            o_ref[pl.ds(me * mq + b * mb, mb), :] = mine

        for b in range(N_BLK):
            for o in _WAIT_ORDER:
                ag_rdmas[b, o].wait_recv()
                src_pos = (me - o) % N_DEV
                o_ref[pl.ds(src_pos * mq + b * mb, mb), :] = (
                    ag_comm[b, o - 1].astype(jnp.float32)
                )

        for b in range(N_BLK):
            for o in _ORDER:
                rs_rdmas[b, o].wait_send()
                ag_rdmas[b, o].wait_send()

    return pl.pallas_call(
        body,
        out_shape=jax.ShapeDtypeStruct((m, n), jnp.float32),
        in_specs=[pl.BlockSpec(memory_space=pltpu.VMEM)] * 3,
        out_specs=pl.BlockSpec(memory_space=pltpu.VMEM),
        scratch_shapes=[
            pltpu.VMEM((m, n), jnp.bfloat16),
            pltpu.VMEM((N_BLK, N_DEV - 1, mb, n), jnp.bfloat16),
            pltpu.VMEM((mq, n), jnp.bfloat16),
            pltpu.VMEM((N_BLK, N_DEV - 1, mb, n), jnp.bfloat16),
            pltpu.SemaphoreType.DMA((N_BLK, N_DEV - 1)),
            pltpu.SemaphoreType.DMA((N_BLK, N_DEV - 1)),
            pltpu.SemaphoreType.DMA((N_BLK, N_DEV - 1)),
            pltpu.SemaphoreType.DMA((N_BLK, N_DEV - 1)),
        ],
        compiler_params=pltpu.CompilerParams(collective_id=0),
    )(partial, resid, gamma2)


# device time: 15321 ns/iter; 1.0223x vs baseline; 1.0087x over previous
import jax
import jax.numpy as jnp
from jax import lax
from jax.experimental import pallas as pl
from jax.experimental.pallas import tpu as pltpu

N_DEV = 4
N_BLK = 4
_ORDER = (2, 1, 3)
_WAIT_ORDER = (1, 3, 2)


def kernel(partial, resid, gamma):
    _, m, n = partial.shape
    mq = m // N_DEV
    mb = mq // N_BLK
    gamma2 = gamma.reshape(1, n)

    def body(
        p_ref, r_ref, g_ref, o_ref,
        sendbuf, rs_comm,
        rs_send_sems, rs_recv_sems, ag_send_sems, ag_recv_sems,
    ):
        me = lax.axis_index("i")

        barrier_sem = pltpu.get_barrier_semaphore()
        for o in range(1, N_DEV):
            pl.semaphore_signal(
                barrier_sem, inc=1,
                device_id=((me + o) % N_DEV,),
                device_id_type=pl.DeviceIdType.MESH,
            )
        sendbuf[...] = p_ref[0].astype(jnp.bfloat16)
        pl.semaphore_wait(barrier_sem, N_DEV - 1)

        rs_rdmas = {}
        for b in range(N_BLK):
            for o in _ORDER:
                dst = (me + o) % N_DEV
                rdma = pltpu.make_async_remote_copy(
                    src_ref=sendbuf.at[pl.ds(dst * mq + b * mb, mb), :],
                    dst_ref=rs_comm.at[b, o - 1],
                    send_sem=rs_send_sems.at[b, o - 1],
                    recv_sem=rs_recv_sems.at[b, o - 1],
                    device_id=(dst,),
                    device_id_type=pl.DeviceIdType.MESH,
                )
                rdma.start()
                rs_rdmas[b, o] = rdma

        ag_rdmas = {}
        for b in range(N_BLK):
            rows = pl.ds(me * mq + b * mb, mb)
            y = p_ref[0, rows, :] + r_ref[rows, :]
            for o in _WAIT_ORDER:
                rs_rdmas[b, o].wait_recv()
                y = y + rs_comm[b, o - 1].astype(jnp.float32)
            rms = jnp.sqrt(jnp.mean(y * y, axis=-1, keepdims=True) + 1e-6)
            o_ref[rows, :] = (y / rms * g_ref[...]).astype(jnp.bfloat16)
            for o in _ORDER:
                dst = (me + o) % N_DEV
                rdma = pltpu.make_async_remote_copy(
                    src_ref=o_ref.at[rows, :],
                    dst_ref=o_ref.at[rows, :],
                    send_sem=ag_send_sems.at[b, o - 1],
                    recv_sem=ag_recv_sems.at[b, o - 1],
                    device_id=(dst,),
                    device_id_type=pl.DeviceIdType.MESH,
                )
                rdma.start()
                ag_rdmas[b, o] = rdma

        for b in range(N_BLK):
            for o in _WAIT_ORDER:
                ag_rdmas[b, o].wait_recv()
        for b in range(N_BLK):
            for o in _ORDER:
                rs_rdmas[b, o].wait_send()
                ag_rdmas[b, o].wait_send()

    return pl.pallas_call(
        body,
        out_shape=jax.ShapeDtypeStruct((m, n), jnp.bfloat16),
        in_specs=[pl.BlockSpec(memory_space=pltpu.VMEM)] * 3,
        out_specs=pl.BlockSpec(memory_space=pltpu.VMEM),
        scratch_shapes=[
            pltpu.VMEM((m, n), jnp.bfloat16),
            pltpu.VMEM((N_BLK, N_DEV - 1, mb, n), jnp.bfloat16),
            pltpu.SemaphoreType.DMA((N_BLK, N_DEV - 1)),
            pltpu.SemaphoreType.DMA((N_BLK, N_DEV - 1)),
            pltpu.SemaphoreType.DMA((N_BLK, N_DEV - 1)),
            pltpu.SemaphoreType.DMA((N_BLK, N_DEV - 1)),
        ],
        compiler_params=pltpu.CompilerParams(collective_id=0),
    )(partial, resid, gamma2)
